# baseline (speedup 1.0000x reference)
import functools

import jax
import jax.numpy as jnp
from jax import lax
from jax.experimental import pallas as pl
from jax.experimental.pallas import tpu as pltpu
from jax.experimental.pallas import tpu_sc as plsc

_CHUNK = 128
_NBUF = 5
_K = 3


def _embed_lookup(table, idx2d, n_rows, n_workers):
    rows_per_w = n_rows // n_workers
    chunks_per_w = rows_per_w // _CHUNK
    n_outer = chunks_per_w // _NBUF
    d = table.shape[1]
    mesh = plsc.VectorSubcoreMesh(core_axis_name="c", subcore_axis_name="s")
    num_cores = mesh.num_cores

    @functools.partial(
        pl.kernel,
        out_type=jax.ShapeDtypeStruct((n_rows, d), table.dtype),
        mesh=mesh,
        scratch_types=[
            pltpu.VMEM((chunks_per_w, _CHUNK), jnp.int32),
            pltpu.VMEM((_NBUF, _CHUNK, d), table.dtype),
            pltpu.VMEM_SHARED(table.shape, table.dtype),
            [pltpu.SemaphoreType.DMA] * _NBUF,
            [pltpu.SemaphoreType.DMA] * _NBUF,
        ],
    )
    def run(table_hbm, idx_hbm, out_hbm, idx_v, rows_v, table_sp, gsems,
            ssems):
        sid = lax.axis_index("s")
        wid = sid * num_cores + lax.axis_index("c")
        base = wid * rows_per_w

        @pl.when(sid == 0)
        def _():
            pltpu.sync_copy(table_hbm, table_sp)

        pltpu.sync_copy(idx_hbm.at[pl.ds(wid * chunks_per_w, chunks_per_w)],
                        idx_v)
        plsc.subcore_barrier()

        def gather(chunk, buf):
            return pltpu.make_async_copy(
                table_sp.at[idx_v.at[chunk]], rows_v.at[buf], gsems[buf])

        def scatter(chunk, buf):
            return pltpu.make_async_copy(
                rows_v.at[buf],
                out_hbm.at[pl.ds(base + chunk * _CHUNK, _CHUNK)],
                ssems[buf])

        for b in range(_K):
            gather(b, b).start()

        def step(c, b, *, wait_prev_scatter, do_prefetch):
            gather(c, b).wait()
            scatter(c, b).start()
            bp = (b + _K) % _NBUF
            if wait_prev_scatter:
                scatter(c - (_NBUF - _K), bp).wait()
            if do_prefetch:
                gather(c + _K, bp).start()

        for b in range(_NBUF):
            step(b, b, wait_prev_scatter=b >= _NBUF - _K, do_prefetch=True)

        def outer(t, carry):
            for b in range(_NBUF):
                step(t * _NBUF + b, b, wait_prev_scatter=True,
                     do_prefetch=True)
            return carry

        lax.fori_loop(1, n_outer - 1, outer, 0)

        for b in range(_NBUF):
            c = (n_outer - 1) * _NBUF + b
            step(c, b, wait_prev_scatter=True,
                 do_prefetch=c + _K < chunks_per_w)

        for i in range(_NBUF - _K):
            c = chunks_per_w - (_NBUF - _K) + i
            scatter(0, c % _NBUF).wait()

    return run(table, idx2d)


def kernel(discrete_actions, discrete_embed_weight):
    bsz, seq = discrete_actions.shape
    n_rows = bsz * seq
    idx2d = discrete_actions.astype(jnp.int32).reshape(n_rows // _CHUNK, _CHUNK)
    out = _embed_lookup(discrete_embed_weight, idx2d, n_rows, n_workers=32)
    return out.reshape(bsz, seq, discrete_embed_weight.shape[1])
